# Initial kernel scaffold; baseline (speedup 1.0000x reference)
import jax
import jax.numpy as jnp
from jax.experimental import pallas as pl


def kernel(cls, regr, refi, target_cls, target_regr, target_refi):
    raise NotImplementedError("write your pallas kernel here")


# speedup vs baseline: 16.9310x; 16.9310x over previous
import functools

import jax
import jax.numpy as jnp
from jax.experimental import pallas as pl
from jax.experimental.pallas import tpu as pltpu

_N = 200000
_NPAD = 204800


def _loss_body(c0_ref, c1_ref, y_ref, out_ref):
    c0 = c0_ref[...]
    c1 = c1_ref[...]
    y = y_ref[...]

    m = jnp.maximum(c0, c1)
    sp = jnp.log1p(jnp.exp(-jnp.abs(c0 - c1)))
    chosen = jnp.where(y == 1, c1, c0)
    loss = jnp.maximum(m - chosen + sp, 0.0)

    pos = y == 1
    neg = y == 0
    onesf = jnp.ones_like(loss)
    zerosf = jnp.zeros_like(loss)
    n_pos = jnp.sum(jnp.where(pos, onesf, zerosf))
    n_neg = jnp.sum(jnp.where(neg, onesf, zerosf))
    pos_sum = jnp.sum(jnp.where(pos, loss, zerosf))
    neg_sum = jnp.sum(jnp.where(neg, loss, zerosf))

    out_ref[0] = pos_sum / n_pos + neg_sum / n_neg

    @pl.when(n_neg > 3.0 * n_pos)
    def _rare():
        k = 3.0 * n_pos
        negloss = jnp.where(neg, loss, -1.0)
        bits = jax.lax.bitcast_convert_type(negloss, jnp.int32)
        ki = k.astype(jnp.int32)

        def step(_, lohi):
            lo, hi = lohi
            mid = (lo + hi) // 2
            cnt = jnp.sum(jnp.where(bits >= mid, onesf, zerosf)).astype(
                jnp.int32)
            take = cnt >= ki
            return jnp.where(take, mid, lo), jnp.where(take, hi, mid)

        lo0 = jnp.int32(0)
        hi0 = jnp.int32(0x7F800000)
        lo, _ = jax.lax.fori_loop(0, 31, step, (lo0, hi0))
        thr = jax.lax.bitcast_convert_type(lo, jnp.float32)
        gt = bits > lo
        cnt_gt = jnp.sum(jnp.where(gt, onesf, zerosf))
        sum_gt = jnp.sum(jnp.where(gt, negloss, zerosf))
        topk_sum = sum_gt + (k - cnt_gt) * thr
        out_ref[0] = pos_sum / n_pos + topk_sum / k


@functools.partial(jax.jit, static_argnames=())
def _rpn_cls_loss(c0, c1, y):
    out = pl.pallas_call(
        _loss_body,
        out_shape=jax.ShapeDtypeStruct((1,), jnp.float32),
        in_specs=[
            pl.BlockSpec(memory_space=pltpu.VMEM),
            pl.BlockSpec(memory_space=pltpu.VMEM),
            pl.BlockSpec(memory_space=pltpu.VMEM),
        ],
        out_specs=pl.BlockSpec(memory_space=pltpu.SMEM),
    )(c0, c1, y)
    return out[0]


def kernel(cls, regr, refi, target_cls, target_regr, target_refi):
    c = cls[0]
    y = target_cls[0, 0]
    pad = _NPAD - _N
    c0 = jnp.pad(c[:, 0], (0, pad)).reshape(1600, 128)
    c1 = jnp.pad(c[:, 1], (0, pad)).reshape(1600, 128)
    yp = jnp.pad(y, (0, pad), constant_values=2).reshape(1600, 128)
    return _rpn_cls_loss(c0, c1, yp)
